# Initial kernel scaffold; baseline (speedup 1.0000x reference)
import jax
import jax.numpy as jnp
from jax.experimental import pallas as pl


def kernel(data, token_table, pos_table):
    raise NotImplementedError("write your pallas kernel here")


# speedup vs baseline: 7.6926x; 7.6926x over previous
import functools

import jax
import jax.numpy as jnp
from jax import lax
from jax.experimental import pallas as pl
from jax.experimental.pallas import tpu as pltpu
from jax.experimental.pallas import tpu_sc as plsc

VOCAB_DIM = 100000
SEQ_LEN = 200
D_MODEL = 128
BATCH = 1024

NC = 2
NS = 16
NW = NC * NS
ROWS_PER_W = BATCH // NW
HALF = SEQ_LEN // 2


def _sc_body(data_hbm, tok_hbm, pos_hbm, out_hbm, idx_v, rows_v, pos_v, sem):
    wid = lax.axis_index("s") * NC + lax.axis_index("c")
    pltpu.sync_copy(pos_hbm, pos_v)

    def chunk(g, carry):
        r = wid * ROWS_PER_W + g
        pltpu.sync_copy(data_hbm.at[r], idx_v)
        cp0 = pltpu.async_copy(tok_hbm.at[idx_v.at[0]],
                               rows_v.at[pl.ds(0, HALF)], sem)
        cp1 = pltpu.async_copy(tok_hbm.at[idx_v.at[1]],
                               rows_v.at[pl.ds(HALF, HALF)], sem)
        cp0.wait()
        cp1.wait()

        def addrow(i, c2):
            for j in range(D_MODEL // 16):
                sl = pl.ds(j * 16, 16)
                rows_v[i, sl] = rows_v[i, sl] + pos_v[i, sl]
            return c2

        lax.fori_loop(0, SEQ_LEN, addrow, 0)
        pltpu.sync_copy(rows_v, out_hbm.at[r])
        return carry

    lax.fori_loop(0, ROWS_PER_W, chunk, 0)


def kernel(data, token_table, pos_table):
    data3 = data.reshape(BATCH, 2, HALF).astype(jnp.int32)
    mesh = plsc.VectorSubcoreMesh(core_axis_name="c", subcore_axis_name="s")
    run = functools.partial(
        pl.kernel,
        out_type=jax.ShapeDtypeStruct((BATCH, SEQ_LEN, D_MODEL), jnp.float32),
        mesh=mesh,
        scratch_types=[
            pltpu.VMEM((2, HALF), jnp.int32),
            pltpu.VMEM((SEQ_LEN, D_MODEL), jnp.float32),
            pltpu.VMEM((SEQ_LEN, D_MODEL), jnp.float32),
            pltpu.SemaphoreType.DMA,
        ],
    )(_sc_body)
    return run(data3, token_table, pos_table)


# speedup vs baseline: 10.6838x; 1.3888x over previous
import functools

import jax
import jax.numpy as jnp
from jax import lax
from jax.experimental import pallas as pl
from jax.experimental.pallas import tpu as pltpu
from jax.experimental.pallas import tpu_sc as plsc

VOCAB_DIM = 100000
SEQ_LEN = 200
D_MODEL = 128
BATCH = 1024

NC = 2
NS = 16
NW = NC * NS
ROWS_PER_W = BATCH // NW
NPAIR = ROWS_PER_W // 2
HALF = SEQ_LEN // 2


def _sc_body(data_hbm, tok_hbm, pos_hbm, out_hbm,
             idx0, idx1, rows0, rows1, pos_v, gsem0, gsem1, osem0, osem1):
    wid = lax.axis_index("s") * NC + lax.axis_index("c")
    base = wid * ROWS_PER_W
    idx_v = (idx0, idx1)
    rows_v = (rows0, rows1)
    gsem = (gsem0, gsem1)
    osem = (osem0, osem1)

    pltpu.sync_copy(pos_hbm, pos_v)

    def issue_gather(g, b):
        pltpu.sync_copy(data_hbm.at[base + g], idx_v[b])
        pltpu.async_copy(tok_hbm.at[idx_v[b].at[0]],
                         rows_v[b].at[pl.ds(0, HALF)], gsem[b])
        pltpu.async_copy(tok_hbm.at[idx_v[b].at[1]],
                         rows_v[b].at[pl.ds(HALF, HALF)], gsem[b])

    def wait_gather(b):
        pltpu.make_async_copy(tok_hbm.at[idx_v[b].at[0]],
                              rows_v[b].at[pl.ds(0, HALF)], gsem[b]).wait()
        pltpu.make_async_copy(tok_hbm.at[idx_v[b].at[1]],
                              rows_v[b].at[pl.ds(HALF, HALF)], gsem[b]).wait()

    def wait_out(b):
        pltpu.make_async_copy(rows_v[b], out_hbm.at[base], osem[b]).wait()

    def add_pos(b):
        def addrow(i, c2):
            for j in range(D_MODEL // 16):
                sl = pl.ds(j * 16, 16)
                rows_v[b][i, sl] = rows_v[b][i, sl] + pos_v[i, sl]
            return c2
        lax.fori_loop(0, SEQ_LEN, addrow, 0)

    issue_gather(0, 0)

    def pair(p, carry):
        wait_gather(0)

        @pl.when(p >= 1)
        def _():
            wait_out(1)
        issue_gather(2 * p + 1, 1)
        add_pos(0)
        pltpu.async_copy(rows_v[0], out_hbm.at[base + 2 * p], osem[0])

        wait_gather(1)
        wait_out(0)

        @pl.when(p < NPAIR - 1)
        def _():
            issue_gather(2 * p + 2, 0)
        add_pos(1)
        pltpu.async_copy(rows_v[1], out_hbm.at[base + 2 * p + 1], osem[1])
        return carry

    lax.fori_loop(0, NPAIR, pair, 0)
    wait_out(1)


def kernel(data, token_table, pos_table):
    data3 = data.reshape(BATCH, 2, HALF).astype(jnp.int32)
    mesh = plsc.VectorSubcoreMesh(core_axis_name="c", subcore_axis_name="s")
    run = functools.partial(
        pl.kernel,
        out_type=jax.ShapeDtypeStruct((BATCH, SEQ_LEN, D_MODEL), jnp.float32),
        mesh=mesh,
        scratch_types=[
            pltpu.VMEM((2, HALF), jnp.int32),
            pltpu.VMEM((2, HALF), jnp.int32),
            pltpu.VMEM((SEQ_LEN, D_MODEL), jnp.float32),
            pltpu.VMEM((SEQ_LEN, D_MODEL), jnp.float32),
            pltpu.VMEM((SEQ_LEN, D_MODEL), jnp.float32),
            pltpu.SemaphoreType.DMA,
            pltpu.SemaphoreType.DMA,
            pltpu.SemaphoreType.DMA,
            pltpu.SemaphoreType.DMA,
        ],
    )(_sc_body)
    return run(data3, token_table, pos_table)


# speedup vs baseline: 10.6895x; 1.0005x over previous
import functools

import jax
import jax.numpy as jnp
from jax import lax
from jax.experimental import pallas as pl
from jax.experimental.pallas import tpu as pltpu
from jax.experimental.pallas import tpu_sc as plsc

VOCAB_DIM = 100000
SEQ_LEN = 200
D_MODEL = 128
BATCH = 1024

NC = 2
NS = 16
NW = NC * NS
ROWS_PER_W = BATCH // NW
NPAIR = ROWS_PER_W // 2
HALF = SEQ_LEN // 2


def _sc_body(data_hbm, tok_hbm, pos_hbm, out_hbm,
             idx0, idx1, rows0, rows1, pos_v, gsem0, gsem1, osem0, osem1):
    wid = lax.axis_index("s") * NC + lax.axis_index("c")
    base = wid * ROWS_PER_W
    idx_v = (idx0, idx1)
    rows_v = (rows0, rows1)
    gsem = (gsem0, gsem1)
    osem = (osem0, osem1)

    pltpu.sync_copy(pos_hbm, pos_v)

    def issue_gather(g, b):
        pltpu.sync_copy(data_hbm.at[base + g], idx_v[b])
        pltpu.async_copy(tok_hbm.at[idx_v[b].at[0]],
                         rows_v[b].at[pl.ds(0, HALF)], gsem[b])
        pltpu.async_copy(tok_hbm.at[idx_v[b].at[1]],
                         rows_v[b].at[pl.ds(HALF, HALF)], gsem[b])

    def wait_gather(b):
        pltpu.make_async_copy(tok_hbm.at[idx_v[b].at[0]],
                              rows_v[b].at[pl.ds(0, HALF)], gsem[b]).wait()
        pltpu.make_async_copy(tok_hbm.at[idx_v[b].at[1]],
                              rows_v[b].at[pl.ds(HALF, HALF)], gsem[b]).wait()

    def wait_out(b):
        pltpu.make_async_copy(rows_v[b], out_hbm.at[base], osem[b]).wait()

    def add_pos(b):
        @plsc.parallel_loop(0, SEQ_LEN, step=1, unroll=4)
        def addrow(i):
            for j in range(D_MODEL // 16):
                sl = pl.ds(j * 16, 16)
                rows_v[b][i, sl] = rows_v[b][i, sl] + pos_v[i, sl]

    issue_gather(0, 0)

    def pair(p, carry):
        wait_gather(0)

        @pl.when(p >= 1)
        def _():
            wait_out(1)
        issue_gather(2 * p + 1, 1)
        add_pos(0)
        pltpu.async_copy(rows_v[0], out_hbm.at[base + 2 * p], osem[0])

        wait_gather(1)
        wait_out(0)

        @pl.when(p < NPAIR - 1)
        def _():
            issue_gather(2 * p + 2, 0)
        add_pos(1)
        pltpu.async_copy(rows_v[1], out_hbm.at[base + 2 * p + 1], osem[1])
        return carry

    lax.fori_loop(0, NPAIR, pair, 0)
    wait_out(1)


def kernel(data, token_table, pos_table):
    data3 = data.reshape(BATCH, 2, HALF).astype(jnp.int32)
    mesh = plsc.VectorSubcoreMesh(core_axis_name="c", subcore_axis_name="s")
    run = functools.partial(
        pl.kernel,
        out_type=jax.ShapeDtypeStruct((BATCH, SEQ_LEN, D_MODEL), jnp.float32),
        mesh=mesh,
        scratch_types=[
            pltpu.VMEM((2, HALF), jnp.int32),
            pltpu.VMEM((2, HALF), jnp.int32),
            pltpu.VMEM((SEQ_LEN, D_MODEL), jnp.float32),
            pltpu.VMEM((SEQ_LEN, D_MODEL), jnp.float32),
            pltpu.VMEM((SEQ_LEN, D_MODEL), jnp.float32),
            pltpu.SemaphoreType.DMA,
            pltpu.SemaphoreType.DMA,
            pltpu.SemaphoreType.DMA,
            pltpu.SemaphoreType.DMA,
        ],
    )(_sc_body)
    return run(data3, token_table, pos_table)
